# baseline (speedup 1.0000x reference)
import functools

import jax
import jax.numpy as jnp
import numpy as np
from jax import lax
from jax.experimental import pallas as pl
from jax.experimental.pallas import tpu as pltpu
from jax.experimental.pallas import tpu_sc as plsc

_NODES = 10000
_EDGES = 320000
_D = 128
_DW = _D // 2

_NC = 2
_NS = 16
_NW = _NC * _NS
_EPW = _EDGES // _NW
_B = 40
_NCH = _EPW // _B
_NG = 5
_CPG = _NCH // _NG
_EPG = _CPG * _B
_PAIRS = _CPG // 2
_NPAD = 10112
_RPS = _NPAD // _NS
_ZR = 79

_PERM = np.empty(_D, dtype=np.int32)
for _k in range(_D // 32):
    for _l in range(16):
        _PERM[32 * _k + _l] = 32 * _k + 2 * _l
        _PERM[32 * _k + 16 + _l] = 32 * _k + 2 * _l + 1


def _sc_segment_sum(src3, dst3, featw):
    mesh = plsc.VectorSubcoreMesh(core_axis_name="c", subcore_axis_name="s")

    @functools.partial(
        pl.kernel,
        out_type=[
            jax.ShapeDtypeStruct((_NC, _NPAD, _D), jnp.float32),
            jax.ShapeDtypeStruct((_NC, _NPAD), jnp.float32),
        ],
        mesh=mesh,
        compiler_params=pltpu.CompilerParams(needs_layout_passes=False, use_tc_tiling_on_sc=False),
        scratch_types=[
            pltpu.VMEM((_EPG,), jnp.int32),
            pltpu.VMEM((_EPG,), jnp.int32),
            pltpu.VMEM((_EPG,), jnp.int32),
            pltpu.VMEM((_EPG,), jnp.int32),
            pltpu.VMEM((2, _B, _DW), jnp.int32),
            pltpu.VMEM((2, _B, _D), jnp.float32),
            pltpu.VMEM((_B,), jnp.float32),
            pltpu.VMEM((_ZR, _D), jnp.float32),
            pltpu.VMEM((_RPS,), jnp.float32),
            pltpu.VMEM_SHARED((_NPAD, _D), jnp.float32),
            pltpu.VMEM_SHARED((_NPAD,), jnp.float32),
            pltpu.SemaphoreType.DMA,
            pltpu.SemaphoreType.DMA,
            pltpu.SemaphoreType.DMA,
            pltpu.SemaphoreType.DMA,
            pltpu.SemaphoreType.DMA,
            pltpu.SemaphoreType.DMA,
            pltpu.SemaphoreType.DMA,
        ],
    )
    def sc(src_hbm, dst3_hbm, feat_hbm, acc_out, deg_out,
           sf0, sf1, df0, df1, rbf_v, rows_v, ones_v, zf_v, zd_v,
           acc_sh, deg_sh, g0, g1, s0, s1, d0, d1, isem):
        c = lax.axis_index("c")
        s = lax.axis_index("s")
        wid = s * _NC + c
        gsem = (g0, g1)
        ssem = (s0, s1)
        dsem = (d0, d1)
        sfb = (sf0, sf1)
        dfb = (df0, df1)

        zero16f = jnp.zeros((16,), jnp.float32)
        one16f = jnp.ones((16,), jnp.float32)

        for k in range(_B // 16):
            ones_v[pl.ds(16 * k, 16)] = one16f
        ones_v[pl.ds(_B - 16, 16)] = one16f

        def init_zf(i, carry):
            for k in range(_D // 16):
                zf_v[i, pl.ds(16 * k, 16)] = zero16f
            return carry

        lax.fori_loop(0, _ZR, init_zf, 0)

        def init_zd(i, carry):
            zd_v[pl.ds(16 * i, 16)] = zero16f
            return carry

        lax.fori_loop(0, _RPS // 16, init_zd, 0)

        def stage_start(g, p):
            pltpu.async_copy(src_hbm.at[wid, g, 0], sfb[p], isem)
            pltpu.async_copy(dst3_hbm.at[wid, g, 0], dfb[p], isem)

        def stage_wait(g, p):
            pltpu.make_async_copy(src_hbm.at[wid, g, 0], sfb[p], isem).wait()
            pltpu.make_async_copy(dst3_hbm.at[wid, g, 0], dfb[p], isem).wait()

        def gather_start(p, j, slot):
            pltpu.async_copy(feat_hbm.at[sfb[p].at[pl.ds(j * _B, _B)]],
                             rbf_v.at[slot], gsem[slot])

        def gather_wait(p, j, slot):
            pltpu.make_async_copy(feat_hbm.at[sfb[p].at[pl.ds(j * _B, _B)]],
                                  rbf_v.at[slot], gsem[slot]).wait()

        def convert(slot):
            def conv_row(r, carry):
                for k in range(_DW // 16):
                    x = rbf_v[slot, r, pl.ds(16 * k, 16)]
                    ab = plsc.bitcast(x, jnp.bfloat16)
                    a, b = plsc.unpack(ab, format=plsc.PackFormat.INTERLEAVED)
                    rows_v[slot, r, pl.ds(32 * k, 16)] = a
                    rows_v[slot, r, pl.ds(32 * k + 16, 16)] = b
                return carry

            lax.fori_loop(0, _B, conv_row, 0)

        def scatter_start(p, j, slot):
            pltpu.async_copy(rows_v.at[slot],
                             acc_sh.at[dfb[p].at[pl.ds(j * _B, _B)]],
                             ssem[slot], add=True)

        def scatter_wait(p, j, slot):
            pltpu.make_async_copy(rows_v.at[slot],
                                  acc_sh.at[dfb[p].at[pl.ds(j * _B, _B)]],
                                  ssem[slot]).wait()

        def deg_start(p, j, slot):
            pltpu.async_copy(ones_v, deg_sh.at[dfb[p].at[pl.ds(j * _B, _B)]],
                             dsem[slot], add=True)

        def deg_wait(p, j, slot):
            pltpu.make_async_copy(ones_v,
                                  deg_sh.at[dfb[p].at[pl.ds(j * _B, _B)]],
                                  dsem[slot]).wait()

        stage_start(0, 0)
        stage_wait(0, 0)
        gather_start(0, 0, 0)
        gather_start(0, 1, 1)

        for k in range(_RPS // _ZR):
            pltpu.sync_copy(zf_v, acc_sh.at[pl.ds(s * _RPS + k * _ZR, _ZR)])
        pltpu.sync_copy(zd_v, deg_sh.at[pl.ds(s * _RPS, _RPS)])
        plsc.subcore_barrier()

        for g in range(_NG):
            p = g & 1

            if g + 1 < _NG:
                stage_start(g + 1, 1 - p)

            def make_pair(first_group):
                def pair(i, carry):
                    for b_ in range(2):
                        j = 2 * i + b_
                        slot = b_
                        if first_group:
                            @pl.when(i > 0)
                            def _():
                                scatter_wait(p, j, slot)
                                deg_wait(p, j, slot)
                        else:
                            scatter_wait(p, j, slot)
                            deg_wait(p, j, slot)
                        gather_wait(p, j, slot)
                        convert(slot)

                        @pl.when(j + 2 < _CPG)
                        def _():
                            gather_start(p, j + 2, slot)

                        scatter_start(p, j, slot)
                        deg_start(p, j, slot)
                    return carry

                return pair

            lax.fori_loop(0, _PAIRS, make_pair(g == 0), 0)

            if g + 1 < _NG:
                stage_wait(g + 1, 1 - p)
                gather_start(1 - p, 0, 0)
                gather_start(1 - p, 1, 1)

        lastp = (_NG - 1) & 1
        for slot in (0, 1):
            scatter_wait(lastp, _CPG - 2 + slot, slot)
            deg_wait(lastp, _CPG - 2 + slot, slot)
        plsc.subcore_barrier()

        pltpu.sync_copy(acc_sh.at[pl.ds(s * _RPS, _RPS)],
                        acc_out.at[c, pl.ds(s * _RPS, _RPS)])
        pltpu.sync_copy(deg_sh.at[pl.ds(s * _RPS, _RPS)],
                        deg_out.at[c, pl.ds(s * _RPS, _RPS)])

    return sc(src3, dst3, featw)


def _tc_finish(acc2, deg2, Wp, b2):
    def body(acc_ref, deg_ref, w_ref, b_ref, out_ref):
        a = acc_ref[0, :_NODES] + acc_ref[1, :_NODES]
        d = deg_ref[0, :_NODES] + deg_ref[1, :_NODES]
        d = jnp.reshape(jnp.maximum(d, 1.0), (_NODES, 1))
        h = a / d
        y = lax.dot_general(h, w_ref[...], (((1,), (1,)), ((), ())),
                            preferred_element_type=jnp.float32)
        out_ref[...] = jnp.maximum(y + b_ref[...], 0.0)

    return pl.pallas_call(
        body,
        out_shape=jax.ShapeDtypeStruct((_NODES, _D), jnp.float32),
    )(acc2, deg2, Wp, b2)


def kernel(feature, edge_index, W, b):
    src3 = edge_index[0].astype(jnp.int32).reshape(_NW, _NG, 1, _EPG)
    dst3 = edge_index[1].astype(jnp.int32).reshape(_NW, _NG, 1, _EPG)
    fbf = feature.astype(jnp.bfloat16).reshape(_NODES, _DW, 2)
    featw = jax.lax.bitcast_convert_type(fbf, jnp.int32)
    acc2, deg2 = _sc_segment_sum(src3, dst3, featw)
    Wp = W[:, jnp.asarray(_PERM)]
    return _tc_finish(acc2, deg2, Wp, b.reshape(1, _D))


# speedup vs baseline: 1.3135x; 1.3135x over previous
import functools

import jax
import jax.numpy as jnp
from jax import lax
from jax.experimental import pallas as pl
from jax.experimental.pallas import tpu as pltpu
from jax.experimental.pallas import tpu_sc as plsc

_NODES = 10000
_EDGES = 320000
_D = 128

_NC = 2
_NS = 16
_NW = _NC * _NS
_EPW = _EDGES // _NW
_B = 40
_NCH = _EPW // _B
_NG = 1
_CPG = _NCH // _NG
_EPG = _CPG * _B
_PAIRS = _CPG // 2
_NPAD = 10240
_RPS = _NPAD // _NS
_ZR = 32


def _sc_segment_sum(src3, dst3, feature):
    mesh = plsc.VectorSubcoreMesh(core_axis_name="c", subcore_axis_name="s")

    @functools.partial(
        pl.kernel,
        out_type=[
            jax.ShapeDtypeStruct((_NC, _NPAD, _D), jnp.float32),
            jax.ShapeDtypeStruct((_NC, _NPAD), jnp.float32),
        ],
        mesh=mesh,
        scratch_types=[
            pltpu.VMEM((_EPG,), jnp.int32),
            pltpu.VMEM((_EPG,), jnp.int32),
            pltpu.VMEM((2, _B, _D), jnp.float32),
            pltpu.VMEM((_B,), jnp.float32),
            pltpu.VMEM((_ZR, _D), jnp.float32),
            pltpu.VMEM((_RPS,), jnp.float32),
            pltpu.VMEM_SHARED((_NPAD, _D), jnp.float32),
            pltpu.VMEM_SHARED((_NPAD,), jnp.float32),
            pltpu.SemaphoreType.DMA,
            pltpu.SemaphoreType.DMA,
            pltpu.SemaphoreType.DMA,
            pltpu.SemaphoreType.DMA,
            pltpu.SemaphoreType.DMA,
            pltpu.SemaphoreType.DMA,
            pltpu.SemaphoreType.DMA,
        ],
    )
    def sc(src_hbm, dst3_hbm, feat_hbm, acc_out, deg_out,
           sf0, df0, rows_v, ones_v, zf_v, zd_v, acc_sh, deg_sh,
           g0, g1, s0, s1, d0, d1, isem):
        c = lax.axis_index("c")
        s = lax.axis_index("s")
        wid = s * _NC + c
        gsem = (g0, g1)
        ssem = (s0, s1)
        dsem = (d0, d1)
        sfb = (sf0, sf0)
        dfb = (df0, df0)

        zero16f = jnp.zeros((16,), jnp.float32)
        one16f = jnp.ones((16,), jnp.float32)

        for k in range(_B // 16):
            ones_v[pl.ds(16 * k, 16)] = one16f
        ones_v[pl.ds(_B - 16, 16)] = one16f

        def init_zf(i, carry):
            for k in range(_D // 16):
                zf_v[i, pl.ds(16 * k, 16)] = zero16f
            return carry

        lax.fori_loop(0, _ZR, init_zf, 0)

        def init_zd(i, carry):
            zd_v[pl.ds(16 * i, 16)] = zero16f
            return carry

        lax.fori_loop(0, _RPS // 16, init_zd, 0)

        def stage_start(g, p):
            pltpu.async_copy(src_hbm.at[wid, g, 0], sfb[p], isem)
            pltpu.async_copy(dst3_hbm.at[wid, g, 0], dfb[p], isem)

        def stage_wait(g, p):
            pltpu.make_async_copy(src_hbm.at[wid, g, 0], sfb[p], isem).wait()
            pltpu.make_async_copy(dst3_hbm.at[wid, g, 0], dfb[p], isem).wait()

        def gather_start(p, j, slot):
            pltpu.async_copy(feat_hbm.at[sfb[p].at[pl.ds(j * _B, _B)]],
                             rows_v.at[slot], gsem[slot])

        def gather_wait(p, j, slot):
            pltpu.make_async_copy(feat_hbm.at[sfb[p].at[pl.ds(j * _B, _B)]],
                                  rows_v.at[slot], gsem[slot]).wait()

        def scatter_start(p, j, slot):
            pltpu.async_copy(rows_v.at[slot],
                             acc_sh.at[dfb[p].at[pl.ds(j * _B, _B)]],
                             ssem[slot], add=True)

        def scatter_wait(p, j, slot):
            pltpu.make_async_copy(rows_v.at[slot],
                                  acc_sh.at[dfb[p].at[pl.ds(j * _B, _B)]],
                                  ssem[slot]).wait()

        def deg_start(p, j, slot):
            pltpu.async_copy(ones_v, deg_sh.at[dfb[p].at[pl.ds(j * _B, _B)]],
                             dsem[slot], add=True)

        def deg_wait(p, j, slot):
            pltpu.make_async_copy(ones_v,
                                  deg_sh.at[dfb[p].at[pl.ds(j * _B, _B)]],
                                  dsem[slot]).wait()

        stage_start(0, 0)
        stage_wait(0, 0)
        gather_start(0, 0, 0)
        gather_start(0, 1, 1)

        for k in range(_RPS // _ZR):
            pltpu.sync_copy(zf_v, acc_sh.at[pl.ds(s * _RPS + k * _ZR, _ZR)])
        pltpu.sync_copy(zd_v, deg_sh.at[pl.ds(s * _RPS, _RPS)])
        plsc.subcore_barrier()

        for g in range(_NG):
            p = g & 1

            if g + 1 < _NG:
                stage_start(g + 1, 1 - p)

            def pair(i, carry):
                j0 = 2 * i
                j1 = 2 * i + 1
                gather_wait(p, j0, 0)
                scatter_start(p, j0, 0)
                deg_start(p, j0, 0)
                gather_wait(p, j1, 1)
                scatter_start(p, j1, 1)
                deg_start(p, j1, 1)

                scatter_wait(p, j0, 0)
                deg_wait(p, j0, 0)

                @pl.when(j0 + 2 < _CPG)
                def _():
                    gather_start(p, j0 + 2, 0)

                scatter_wait(p, j1, 1)
                deg_wait(p, j1, 1)

                @pl.when(j1 + 2 < _CPG)
                def _():
                    gather_start(p, j1 + 2, 1)

                return carry

            lax.fori_loop(0, _PAIRS, pair, 0)

            if g + 1 < _NG:
                stage_wait(g + 1, 1 - p)
                gather_start(1 - p, 0, 0)
                gather_start(1 - p, 1, 1)

        plsc.subcore_barrier()

        pltpu.sync_copy(acc_sh.at[pl.ds(s * _RPS, _RPS)],
                        acc_out.at[c, pl.ds(s * _RPS, _RPS)])
        pltpu.sync_copy(deg_sh.at[pl.ds(s * _RPS, _RPS)],
                        deg_out.at[c, pl.ds(s * _RPS, _RPS)])

    return sc(src3, dst3, feature)


def _tc_finish(acc2, deg2, W, b2):
    def body(acc_ref, deg_ref, w_ref, b_ref, out_ref):
        a = acc_ref[0, :_NODES] + acc_ref[1, :_NODES]
        d = deg_ref[0] + deg_ref[1]
        d = jnp.reshape(jnp.maximum(d[:_NODES], 1.0), (_NODES, 1))
        h = a / d
        y = lax.dot_general(h, w_ref[...], (((1,), (1,)), ((), ())),
                            preferred_element_type=jnp.float32)
        out_ref[...] = jnp.maximum(y + b_ref[...], 0.0)

    return pl.pallas_call(
        body,
        out_shape=jax.ShapeDtypeStruct((_NODES, _D), jnp.float32),
    )(acc2, deg2, W, b2)


def kernel(feature, edge_index, W, b):
    src3 = edge_index[0].astype(jnp.int32).reshape(_NW, _NG, 1, _EPG)
    dst3 = edge_index[1].astype(jnp.int32).reshape(_NW, _NG, 1, _EPG)
    acc2, deg2 = _sc_segment_sum(src3, dst3, feature)
    return _tc_finish(acc2, deg2, W, b.reshape(1, _D))


# speedup vs baseline: 1.3158x; 1.0017x over previous
import functools

import jax
import jax.numpy as jnp
from jax import lax
from jax.experimental import pallas as pl
from jax.experimental.pallas import tpu as pltpu
from jax.experimental.pallas import tpu_sc as plsc

_NODES = 10000
_EDGES = 320000
_D = 128

_NC = 2
_NS = 16
_NW = _NC * _NS
_EPW = _EDGES // _NW
_B = 40
_NCH = _EPW // _B
_NG = 1
_CPG = _NCH // _NG
_EPG = _CPG * _B
_PAIRS = _CPG // 2
_NPAD = 10240
_RPS = _NPAD // _NS
_ZR = 32


def _sc_segment_sum(src3, dst3, feature):
    mesh = plsc.VectorSubcoreMesh(core_axis_name="c", subcore_axis_name="s")

    @functools.partial(
        pl.kernel,
        out_type=[
            jax.ShapeDtypeStruct((_NC, _NPAD, _D), jnp.float32),
            jax.ShapeDtypeStruct((_NC, _NPAD), jnp.float32),
        ],
        mesh=mesh,
        scratch_types=[
            pltpu.VMEM((_EPG,), jnp.int32),
            pltpu.VMEM((_EPG,), jnp.int32),
            pltpu.VMEM((2, _B, _D), jnp.float32),
            pltpu.VMEM((2 * _B,), jnp.float32),
            pltpu.VMEM((_ZR, _D), jnp.float32),
            pltpu.VMEM((_RPS,), jnp.float32),
            pltpu.VMEM_SHARED((_NPAD, _D), jnp.float32),
            pltpu.VMEM_SHARED((_NPAD,), jnp.float32),
            pltpu.SemaphoreType.DMA,
            pltpu.SemaphoreType.DMA,
            pltpu.SemaphoreType.DMA,
            pltpu.SemaphoreType.DMA,
            pltpu.SemaphoreType.DMA,
            pltpu.SemaphoreType.DMA,
            pltpu.SemaphoreType.DMA,
        ],
    )
    def sc(src_hbm, dst3_hbm, feat_hbm, acc_out, deg_out,
           sf0, df0, rows_v, ones_v, zf_v, zd_v, acc_sh, deg_sh,
           g0, g1, s0, s1, d0, d1, isem):
        c = lax.axis_index("c")
        s = lax.axis_index("s")
        wid = s * _NC + c
        gsem = (g0, g1)
        ssem = (s0, s1)
        dsem = (d0, d1)
        sfb = (sf0, sf0)
        dfb = (df0, df0)

        zero16f = jnp.zeros((16,), jnp.float32)
        one16f = jnp.ones((16,), jnp.float32)

        for k in range(2 * _B // 16):
            ones_v[pl.ds(16 * k, 16)] = one16f

        def init_zf(i, carry):
            for k in range(_D // 16):
                zf_v[i, pl.ds(16 * k, 16)] = zero16f
            return carry

        lax.fori_loop(0, _ZR, init_zf, 0)

        def init_zd(i, carry):
            zd_v[pl.ds(16 * i, 16)] = zero16f
            return carry

        lax.fori_loop(0, _RPS // 16, init_zd, 0)

        def stage_start(g, p):
            pltpu.async_copy(src_hbm.at[wid, g, 0], sfb[p], isem)
            pltpu.async_copy(dst3_hbm.at[wid, g, 0], dfb[p], isem)

        def stage_wait(g, p):
            pltpu.make_async_copy(src_hbm.at[wid, g, 0], sfb[p], isem).wait()
            pltpu.make_async_copy(dst3_hbm.at[wid, g, 0], dfb[p], isem).wait()

        def gather_start(p, j, slot):
            pltpu.async_copy(feat_hbm.at[sfb[p].at[pl.ds(j * _B, _B)]],
                             rows_v.at[slot], gsem[slot])

        def gather_wait(p, j, slot):
            pltpu.make_async_copy(feat_hbm.at[sfb[p].at[pl.ds(j * _B, _B)]],
                                  rows_v.at[slot], gsem[slot]).wait()

        def scatter_start(p, j, slot):
            pltpu.async_copy(rows_v.at[slot],
                             acc_sh.at[dfb[p].at[pl.ds(j * _B, _B)]],
                             ssem[slot], add=True)

        def scatter_wait(p, j, slot):
            pltpu.make_async_copy(rows_v.at[slot],
                                  acc_sh.at[dfb[p].at[pl.ds(j * _B, _B)]],
                                  ssem[slot]).wait()

        def deg_start(p, i):
            pltpu.async_copy(ones_v,
                             deg_sh.at[dfb[p].at[pl.ds(i * 2 * _B, 2 * _B)]],
                             dsem[0], add=True)

        def deg_wait(p, i):
            pltpu.make_async_copy(
                ones_v, deg_sh.at[dfb[p].at[pl.ds(i * 2 * _B, 2 * _B)]],
                dsem[0]).wait()

        stage_start(0, 0)
        stage_wait(0, 0)
        gather_start(0, 0, 0)
        gather_start(0, 1, 1)

        for k in range(_RPS // _ZR):
            pltpu.sync_copy(zf_v, acc_sh.at[pl.ds(s * _RPS + k * _ZR, _ZR)])
        pltpu.sync_copy(zd_v, deg_sh.at[pl.ds(s * _RPS, _RPS)])
        plsc.subcore_barrier()

        for g in range(_NG):
            p = g & 1

            if g + 1 < _NG:
                stage_start(g + 1, 1 - p)

            def pair(i, carry):
                j0 = 2 * i
                j1 = 2 * i + 1
                gather_wait(p, j0, 0)
                scatter_start(p, j0, 0)
                deg_start(p, i)
                gather_wait(p, j1, 1)
                scatter_start(p, j1, 1)

                scatter_wait(p, j0, 0)

                @pl.when(j0 + 2 < _CPG)
                def _():
                    gather_start(p, j0 + 2, 0)

                scatter_wait(p, j1, 1)
                deg_wait(p, i)

                @pl.when(j1 + 2 < _CPG)
                def _():
                    gather_start(p, j1 + 2, 1)

                return carry

            lax.fori_loop(0, _PAIRS, pair, 0)

            if g + 1 < _NG:
                stage_wait(g + 1, 1 - p)
                gather_start(1 - p, 0, 0)
                gather_start(1 - p, 1, 1)

        plsc.subcore_barrier()

        pltpu.sync_copy(acc_sh.at[pl.ds(s * _RPS, _RPS)],
                        acc_out.at[c, pl.ds(s * _RPS, _RPS)])
        pltpu.sync_copy(deg_sh.at[pl.ds(s * _RPS, _RPS)],
                        deg_out.at[c, pl.ds(s * _RPS, _RPS)])

    return sc(src3, dst3, feature)


def _tc_finish(acc2, deg2, W, b2):
    def body(acc_ref, deg_ref, w_ref, b_ref, out_ref):
        a = acc_ref[0, :_NODES] + acc_ref[1, :_NODES]
        d = deg_ref[0] + deg_ref[1]
        d = jnp.reshape(jnp.maximum(d[:_NODES], 1.0), (_NODES, 1))
        h = a / d
        y = lax.dot_general(h, w_ref[...], (((1,), (1,)), ((), ())),
                            preferred_element_type=jnp.float32)
        out_ref[...] = jnp.maximum(y + b_ref[...], 0.0)

    return pl.pallas_call(
        body,
        out_shape=jax.ShapeDtypeStruct((_NODES, _D), jnp.float32),
    )(acc2, deg2, W, b2)


def kernel(feature, edge_index, W, b):
    src3 = edge_index[0].astype(jnp.int32).reshape(_NW, _NG, 1, _EPG)
    dst3 = edge_index[1].astype(jnp.int32).reshape(_NW, _NG, 1, _EPG)
    acc2, deg2 = _sc_segment_sum(src3, dst3, feature)
    return _tc_finish(acc2, deg2, W, b.reshape(1, _D))
